# baseline (device time: 156164 ns/iter reference)
import jax
import jax.numpy as jnp
from jax import lax
from jax.experimental import pallas as pl
from jax.experimental.pallas import tpu as pltpu

N_DEV = 32
N_SUB = 2
N_SLOT = 3


def kernel(x, w_mat, scale_x, scale_w):
    m_glob, k_per = x.shape
    _, n = w_mat.shape
    m_chunk = m_glob // N_DEV
    nh = n // 2
    ns = nh // N_SUB

    def body(x_ref, w_ref, sx_ref, sw_ref, out_ref,
             comm_r, comm_l, part_ref,
             send_r, recv_r, send_l, recv_l, cred_r, cred_l):
        def log_of_pos(p):
            p = lax.rem(p + 2 * N_DEV, N_DEV)
            in_x0 = p < 16
            s = jnp.where(in_x0, p, 31 - p)
            x = jnp.where(in_x0, 0, 1)
            z = s // 4
            t = s - 4 * z
            y = jnp.where(z % 2 == 0, t, 3 - t)
            return 8 * z + 2 * y + jnp.where(y % 2 == 0, x, 1 - x)

        def pos_of_log(i):
            z = i // 8
            r = i - 8 * z
            y = r // 2
            q = r - 2 * y
            x = jnp.where(y % 2 == 0, q, 1 - q)
            s = 4 * z + jnp.where(z % 2 == 0, y, 3 - y)
            return jnp.where(x == 0, s, 31 - s)

        my = lax.axis_index("i")
        pos = pos_of_log(my)
        left = log_of_pos(pos - 1)
        right = log_of_pos(pos + 1)

        barrier = pltpu.get_barrier_semaphore()
        for nbr in (left, right):
            pl.semaphore_signal(
                barrier, inc=1,
                device_id=(nbr,), device_id_type=pl.DeviceIdType.MESH,
            )
        pl.semaphore_wait(barrier, 2)

        part_ref[:, :] = jnp.dot(
            x_ref[:, :].astype(jnp.bfloat16),
            w_ref[:, :].astype(jnp.bfloat16),
            preferred_element_type=jnp.float32)

        def psl(c, lo, hi):
            return part_ref[pl.ds(c * m_chunk, m_chunk), lo:hi]

        def mk(comm, send_sems, recv_sems, j, ss, rs, dev):
            return pltpu.make_async_remote_copy(
                src_ref=comm.at[j, ss],
                dst_ref=comm.at[j, rs],
                send_sem=send_sems.at[j, ss],
                recv_sem=recv_sems.at[j, rs],
                device_id=(dev,),
                device_id_type=pl.DeviceIdType.MESH,
            )

        c0_r = left
        c0_l = right
        for j in range(N_SUB):
            comm_r[j, 0, :, :] = psl(c0_r, j * ns, (j + 1) * ns).astype(
                jnp.bfloat16)
            comm_l[j, 0, :, :] = psl(c0_l, nh + j * ns,
                                     nh + (j + 1) * ns).astype(jnp.bfloat16)

        for h in range(N_DEV - 1):
            ss = h % N_SLOT
            rs = (h + 1) % N_SLOT
            ps = (h - 1) % N_SLOT

            rdmas = []
            for j in range(N_SUB):
                if h >= 2:
                    pl.semaphore_wait(cred_r.at[j], 1)
                rr = mk(comm_r, send_r, recv_r, j, ss, rs, right)
                rr.start()
                if h >= 2:
                    pl.semaphore_wait(cred_l.at[j], 1)
                rl = mk(comm_l, send_l, recv_l, j, ss, rs, left)
                rl.start()
                rdmas.append((rr, rl))

            if h >= 1:
                for j in range(N_SUB):
                    mk(comm_r, send_r, recv_r, j, ps, rs, right).wait_send()
                    mk(comm_l, send_l, recv_l, j, ps, rs, left).wait_send()
                    if h <= N_DEV - 3:
                        pl.semaphore_signal(
                            cred_r.at[j], inc=1,
                            device_id=(left,),
                            device_id_type=pl.DeviceIdType.MESH,
                        )
                        pl.semaphore_signal(
                            cred_l.at[j], inc=1,
                            device_id=(right,),
                            device_id_type=pl.DeviceIdType.MESH,
                        )

            c_r = log_of_pos(pos - h - 2)
            c_l = log_of_pos(pos + h + 2)

            if h < N_DEV - 2:
                for j in range(N_SUB):
                    rr, rl = rdmas[j]
                    rr.wait_recv()
                    comm_r[j, rs, :, :] = (
                        comm_r[j, rs, :, :].astype(jnp.float32)
                        + psl(c_r, j * ns, (j + 1) * ns)
                    ).astype(jnp.bfloat16)
                    rl.wait_recv()
                    comm_l[j, rs, :, :] = (
                        comm_l[j, rs, :, :].astype(jnp.float32)
                        + psl(c_l, nh + j * ns, nh + (j + 1) * ns)
                    ).astype(jnp.bfloat16)
            else:
                scale = sx_ref[0] * sw_ref[0]
                for j in range(N_SUB):
                    rr, rl = rdmas[j]
                    rr.wait_recv()
                    acc = (comm_r[j, rs, :, :].astype(jnp.float32)
                           + psl(c_r, j * ns, (j + 1) * ns))
                    y = acc * scale
                    out_ref[:, j * ns:(j + 1) * ns] = (
                        y * (1.0 / (1.0 + jnp.exp(-y))))
                    rl.wait_recv()
                    acc = (comm_l[j, rs, :, :].astype(jnp.float32)
                           + psl(c_l, nh + j * ns, nh + (j + 1) * ns))
                    y = acc * scale
                    out_ref[:, nh + j * ns:nh + (j + 1) * ns] = (
                        y * (1.0 / (1.0 + jnp.exp(-y))))

        h_last = N_DEV - 2
        for j in range(N_SUB):
            mk(comm_r, send_r, recv_r, j, h_last % N_SLOT,
               (h_last + 1) % N_SLOT, right).wait_send()
            mk(comm_l, send_l, recv_l, j, h_last % N_SLOT,
               (h_last + 1) % N_SLOT, left).wait_send()

    out_shape = jax.ShapeDtypeStruct((m_chunk, n), jnp.float32)
    return pl.pallas_call(
        body,
        out_shape=out_shape,
        in_specs=[
            pl.BlockSpec(memory_space=pltpu.VMEM),
            pl.BlockSpec(memory_space=pltpu.VMEM),
            pl.BlockSpec(memory_space=pltpu.SMEM),
            pl.BlockSpec(memory_space=pltpu.SMEM),
        ],
        out_specs=pl.BlockSpec(memory_space=pltpu.VMEM),
        scratch_shapes=[
            pltpu.VMEM((N_SUB, N_SLOT, m_chunk, ns), jnp.bfloat16),
            pltpu.VMEM((N_SUB, N_SLOT, m_chunk, ns), jnp.bfloat16),
            pltpu.VMEM((m_glob, n), jnp.float32),
            pltpu.SemaphoreType.DMA((N_SUB, N_SLOT)),
            pltpu.SemaphoreType.DMA((N_SUB, N_SLOT)),
            pltpu.SemaphoreType.DMA((N_SUB, N_SLOT)),
            pltpu.SemaphoreType.DMA((N_SUB, N_SLOT)),
            pltpu.SemaphoreType.REGULAR((N_SUB,)),
            pltpu.SemaphoreType.REGULAR((N_SUB,)),
        ],
        compiler_params=pltpu.CompilerParams(
            collective_id=0, vmem_limit_bytes=64 * 1024 * 1024),
    )(x, w_mat, scale_x, scale_w)


# device time: 104228 ns/iter; 1.4983x vs baseline; 1.4983x over previous
import jax
import jax.numpy as jnp
from jax import lax
from jax.experimental import pallas as pl
from jax.experimental.pallas import tpu as pltpu

N_DEV = 32
N_SUB = 4
N_SLOT = 3


def kernel(x, w_mat, scale_x, scale_w):
    m_glob, k_per = x.shape
    _, n = w_mat.shape
    m_chunk = m_glob // N_DEV
    nh = n // 2
    ns = nh // N_SUB

    def body(x_ref, w_ref, sx_ref, sw_ref, out_ref,
             comm_r, comm_l, part_ref,
             send_r, recv_r, send_l, recv_l, cred_r, cred_l):
        def log_of_pos(p):
            p = lax.rem(p + 2 * N_DEV, N_DEV)
            in_x0 = p < 16
            s = jnp.where(in_x0, p, 31 - p)
            x = jnp.where(in_x0, 0, 1)
            z = s // 4
            t = s - 4 * z
            y = jnp.where(z % 2 == 0, t, 3 - t)
            return 8 * z + 2 * y + jnp.where(y % 2 == 0, x, 1 - x)

        def pos_of_log(i):
            z = i // 8
            r = i - 8 * z
            y = r // 2
            q = r - 2 * y
            x = jnp.where(y % 2 == 0, q, 1 - q)
            s = 4 * z + jnp.where(z % 2 == 0, y, 3 - y)
            return jnp.where(x == 0, s, 31 - s)

        my = lax.axis_index("i")
        pos = pos_of_log(my)
        left = log_of_pos(pos - 1)
        right = log_of_pos(pos + 1)

        barrier = pltpu.get_barrier_semaphore()
        for nbr in (left, right):
            pl.semaphore_signal(
                barrier, inc=1,
                device_id=(nbr,), device_id_type=pl.DeviceIdType.MESH,
            )
        pl.semaphore_wait(barrier, 2)

        part_ref[:, :] = jnp.dot(
            x_ref[:, :].astype(jnp.bfloat16),
            w_ref[:, :].astype(jnp.bfloat16),
            preferred_element_type=jnp.float32)

        def psl(c, lo, hi):
            return part_ref[pl.ds(c * m_chunk, m_chunk), lo:hi]

        def mk(comm, send_sems, recv_sems, j, ss, rs, dev):
            return pltpu.make_async_remote_copy(
                src_ref=comm.at[j, ss],
                dst_ref=comm.at[j, rs],
                send_sem=send_sems.at[j, ss],
                recv_sem=recv_sems.at[j, rs],
                device_id=(dev,),
                device_id_type=pl.DeviceIdType.MESH,
            )

        c0_r = left
        c0_l = right
        for j in range(N_SUB):
            comm_r[j, 0, :, :] = psl(c0_r, j * ns, (j + 1) * ns).astype(
                jnp.bfloat16)
            comm_l[j, 0, :, :] = psl(c0_l, nh + j * ns,
                                     nh + (j + 1) * ns).astype(jnp.bfloat16)

        for j in range(N_SUB):
            mk(comm_r, send_r, recv_r, j, 0, 1, right).start()
            mk(comm_l, send_l, recv_l, j, 0, 1, left).start()

        for h in range(1, N_DEV - 1):
            ss = h % N_SLOT
            rs = (h + 1) % N_SLOT
            ps = (h - 1) % N_SLOT
            c_r = log_of_pos(pos - h - 1)
            c_l = log_of_pos(pos + h + 1)
            for j in range(N_SUB):
                for (comm, ssem, rsem, cred, dev, updev, lo) in (
                    (comm_r, send_r, recv_r, cred_r, right, left,
                     j * ns),
                    (comm_l, send_l, recv_l, cred_l, left, right,
                     nh + j * ns),
                ):
                    c = c_r if comm is comm_r else c_l
                    mk(comm, ssem, rsem, j, ps, ss, dev).wait_recv()
                    comm[j, ss, :, :] = (
                        comm[j, ss, :, :].astype(jnp.float32)
                        + psl(c, lo, lo + ns)
                    ).astype(jnp.bfloat16)
                    if h >= 2:
                        pl.semaphore_wait(cred.at[j], 1)
                    mk(comm, ssem, rsem, j, ss, rs, dev).start()
                    mk(comm, ssem, rsem, j, ps, rs, dev).wait_send()
                    if h <= N_DEV - 3:
                        pl.semaphore_signal(
                            cred.at[j], inc=1,
                            device_id=(updev,),
                            device_id_type=pl.DeviceIdType.MESH,
                        )

        h = N_DEV - 1
        ss = h % N_SLOT
        ps = (h - 1) % N_SLOT
        c_r = log_of_pos(pos - h - 1)
        c_l = log_of_pos(pos + h + 1)
        scale = sx_ref[0] * sw_ref[0]
        for j in range(N_SUB):
            mk(comm_r, send_r, recv_r, j, ps, ss, right).wait_recv()
            acc = (comm_r[j, ss, :, :].astype(jnp.float32)
                   + psl(c_r, j * ns, (j + 1) * ns))
            y = acc * scale
            out_ref[:, j * ns:(j + 1) * ns] = (
                y * (1.0 / (1.0 + jnp.exp(-y))))
            mk(comm_l, send_l, recv_l, j, ps, ss, left).wait_recv()
            acc = (comm_l[j, ss, :, :].astype(jnp.float32)
                   + psl(c_l, nh + j * ns, nh + (j + 1) * ns))
            y = acc * scale
            out_ref[:, nh + j * ns:nh + (j + 1) * ns] = (
                y * (1.0 / (1.0 + jnp.exp(-y))))
            mk(comm_r, send_r, recv_r, j, ps, ss, right).wait_send()
            mk(comm_l, send_l, recv_l, j, ps, ss, left).wait_send()

    out_shape = jax.ShapeDtypeStruct((m_chunk, n), jnp.float32)
    return pl.pallas_call(
        body,
        out_shape=out_shape,
        in_specs=[
            pl.BlockSpec(memory_space=pltpu.VMEM),
            pl.BlockSpec(memory_space=pltpu.VMEM),
            pl.BlockSpec(memory_space=pltpu.SMEM),
            pl.BlockSpec(memory_space=pltpu.SMEM),
        ],
        out_specs=pl.BlockSpec(memory_space=pltpu.VMEM),
        scratch_shapes=[
            pltpu.VMEM((N_SUB, N_SLOT, m_chunk, ns), jnp.bfloat16),
            pltpu.VMEM((N_SUB, N_SLOT, m_chunk, ns), jnp.bfloat16),
            pltpu.VMEM((m_glob, n), jnp.float32),
            pltpu.SemaphoreType.DMA((N_SUB, N_SLOT)),
            pltpu.SemaphoreType.DMA((N_SUB, N_SLOT)),
            pltpu.SemaphoreType.DMA((N_SUB, N_SLOT)),
            pltpu.SemaphoreType.DMA((N_SUB, N_SLOT)),
            pltpu.SemaphoreType.REGULAR((N_SUB,)),
            pltpu.SemaphoreType.REGULAR((N_SUB,)),
        ],
        compiler_params=pltpu.CompilerParams(
            collective_id=0, vmem_limit_bytes=64 * 1024 * 1024),
    )(x, w_mat, scale_x, scale_w)
